# baseline (speedup 1.0000x reference)
import functools

import jax
import jax.numpy as jnp
from jax import lax
from jax.experimental import pallas as pl
from jax.experimental.pallas import tpu as pltpu
from jax.experimental.pallas import tpu_sc as plsc

_ROW = 32
_CHUNK = 128
_TBLK = 512


def _tc_body(bt_ref, o_ref):
  o_ref[...] = bt_ref[...].reshape(_ROW, _TBLK).transpose(1, 0)


@functools.cache
def _tc_build(num_boxes: int, m: int):
  nblk = (num_boxes + _TBLK - 1) // _TBLK
  return pl.pallas_call(
      _tc_body,
      grid=(nblk,),
      in_specs=[pl.BlockSpec((1, 2, 16, _TBLK), lambda b, m=m: (m, 0, 0, b))],
      out_specs=pl.BlockSpec((_TBLK, _ROW), lambda b: (b, 0)),
      out_shape=jax.ShapeDtypeStruct((num_boxes, _ROW), jnp.float32),
  )


@functools.cache
def _sc_build(num_models: int, num_boxes: int, batch: int):
  info = plsc.get_sparse_core_info()
  nc, ns = info.num_cores, info.num_subcores
  nw = nc * ns
  b_per_w = batch // nw
  n_chunks = b_per_w // _CHUNK
  mesh = plsc.VectorSubcoreMesh(core_axis_name="c", subcore_axis_name="s")

  @functools.partial(
      pl.kernel,
      mesh=mesh,
      out_type=jax.ShapeDtypeStruct(
          (num_models, nw, n_chunks, _CHUNK, _ROW), jnp.float32),
      scratch_types=[
          pltpu.VMEM((n_chunks, _CHUNK), jnp.int32),
          pltpu.VMEM((num_models, n_chunks, _CHUNK, _ROW), jnp.float32),
          pltpu.SemaphoreType.DMA,
      ],
      compiler_params=pltpu.CompilerParams(use_tc_tiling_on_sc=False),
  )
  def gather(ids_hbm, t0_hbm, t1_hbm, out_hbm, idx_v, rows_v, sem):
    wid = lax.axis_index("s") * nc + lax.axis_index("c")
    base = wid * b_per_w
    for j in range(n_chunks):
      pltpu.sync_copy(ids_hbm.at[pl.ds(base + j * _CHUNK, _CHUNK)],
                      idx_v.at[j])
    copies = []
    for m, table in enumerate((t0_hbm, t1_hbm)):
      for j in range(n_chunks):
        copies.append(
            pltpu.async_copy(table.at[idx_v.at[j]], rows_v.at[m, j], sem))
    for c in copies:
      c.wait()
    for m in range(num_models):
      pltpu.sync_copy(rows_v.at[m], out_hbm.at[m, wid])

  return gather


def kernel(ids, boxes):
  num_models, num_boxes, two, dim = boxes.shape
  batch = ids.shape[0]
  bt = boxes.transpose(0, 2, 3, 1)
  t0 = _tc_build(num_boxes, 0)(bt)
  t1 = _tc_build(num_boxes, 1)(bt)
  out = _sc_build(num_models, num_boxes, batch)(ids.astype(jnp.int32), t0, t1)
  return out.reshape(num_models, batch, two, dim)


# speedup vs baseline: 1.9305x; 1.9305x over previous
import functools

import jax
import jax.numpy as jnp
from jax import lax
from jax.experimental import pallas as pl
from jax.experimental.pallas import tpu as pltpu
from jax.experimental.pallas import tpu_sc as plsc

_ROW = 32
_CHUNK = 128
_TBLK = 2048


def _tc_body(bt_ref, o_ref):
  o_ref[...] = bt_ref[...].reshape(_ROW, _TBLK).transpose(1, 0)


@functools.cache
def _tc_build(num_boxes: int, m: int):
  nblk = (num_boxes + _TBLK - 1) // _TBLK
  return pl.pallas_call(
      _tc_body,
      grid=(nblk,),
      in_specs=[pl.BlockSpec((1, 2, 16, _TBLK), lambda b, m=m: (m, 0, 0, b))],
      out_specs=pl.BlockSpec((_TBLK, _ROW), lambda b: (b, 0)),
      out_shape=jax.ShapeDtypeStruct((num_boxes, _ROW), jnp.float32),
  )


@functools.cache
def _sc_build(num_models: int, num_boxes: int, batch: int):
  info = plsc.get_sparse_core_info()
  nc, ns = info.num_cores, info.num_subcores
  nw = nc * ns
  b_per_w = batch // nw
  n_chunks = b_per_w // _CHUNK
  mesh = plsc.VectorSubcoreMesh(core_axis_name="c", subcore_axis_name="s")

  @functools.partial(
      pl.kernel,
      mesh=mesh,
      out_type=jax.ShapeDtypeStruct(
          (num_models, nw, n_chunks, _CHUNK, _ROW), jnp.float32),
      scratch_types=[
          pltpu.VMEM((n_chunks, _CHUNK), jnp.int32),
          pltpu.VMEM((num_models, n_chunks, _CHUNK, _ROW), jnp.float32),
          pltpu.SemaphoreType.DMA,
      ],
      compiler_params=pltpu.CompilerParams(use_tc_tiling_on_sc=False),
  )
  def gather(ids_hbm, t0_hbm, t1_hbm, out_hbm, idx_v, rows_v, sem):
    wid = lax.axis_index("s") * nc + lax.axis_index("c")
    base = wid * b_per_w
    for j in range(n_chunks):
      pltpu.sync_copy(ids_hbm.at[pl.ds(base + j * _CHUNK, _CHUNK)],
                      idx_v.at[j])
    copies = []
    for m, table in enumerate((t0_hbm, t1_hbm)):
      for j in range(n_chunks):
        copies.append(
            pltpu.async_copy(table.at[idx_v.at[j]], rows_v.at[m, j], sem))
    for c in copies:
      c.wait()
    for m in range(num_models):
      pltpu.sync_copy(rows_v.at[m], out_hbm.at[m, wid])

  return gather


def kernel(ids, boxes):
  num_models, num_boxes, two, dim = boxes.shape
  batch = ids.shape[0]
  bt = boxes.transpose(0, 2, 3, 1)
  t0 = _tc_build(num_boxes, 0)(bt)
  t1 = _tc_build(num_boxes, 1)(bt)
  out = _sc_build(num_models, num_boxes, batch)(ids.astype(jnp.int32), t0, t1)
  return out.reshape(num_models, batch, two, dim)
